# baseline (speedup 1.0000x reference)
import jax
import jax.numpy as jnp
from jax import lax
from jax.experimental import pallas as pl
from jax.experimental.pallas import tpu as pltpu
from jax.experimental.pallas import tpu_sc as plsc

NP = 10240
D = 128
NW = 32
NCHUNK = 160
EPT = NCHUNK * 128
E2P = NW * EPT
IDX_ROWS = E2P // 128


def _deg_body(dst_hbm, w_hbm, out_hbm, dst_v, w_v, deg_s, zb):
    c = lax.axis_index("c")
    s = lax.axis_index("s")
    wid = s * 2 + c
    for q in range(640 // 16):
        zb[pl.ds(q * 16, 16)] = jnp.zeros((16,), jnp.float32)
    pltpu.sync_copy(zb, deg_s.at[pl.ds(s * 640, 640)])
    plsc.subcore_barrier()
    pltpu.sync_copy(dst_hbm.at[pl.ds(wid * NCHUNK, NCHUNK)], dst_v)
    pltpu.sync_copy(w_hbm.at[pl.ds(wid * NCHUNK, NCHUNK)], w_v)

    def body(j, carry):
        pltpu.sync_copy(w_v.at[j], deg_s.at[dst_v.at[j]], add=True)
        return carry

    lax.fori_loop(0, NCHUNK, body, 0)
    plsc.subcore_barrier()
    pltpu.sync_copy(deg_s.at[pl.ds(s * 640, 640)],
                    out_hbm.at[pl.ds(c * NP + s * 640, 640)])


def _degrees(dst2d, w2d):
    return pl.kernel(
        _deg_body,
        out_type=jax.ShapeDtypeStruct((2 * NP,), jnp.float32),
        mesh=plsc.VectorSubcoreMesh(core_axis_name="c", subcore_axis_name="s", num_cores=2, num_subcores=16),
        scratch_types=[
            pltpu.VMEM((NCHUNK, 128), jnp.int32),
            pltpu.VMEM((NCHUNK, 128), jnp.float32),
            pltpu.VMEM_SHARED((NP,), jnp.float32),
            pltpu.VMEM((640,), jnp.float32),
        ],
    )(dst2d, w2d)


NGRP = NCHUNK // 8


def _mp_body(src_hbm, dst_hbm, w_hbm, xs_hbm, out_hbm,
             src_v, dst_v, w_v, rows_v, acc_s, gsem, isem, ssem):
    c = lax.axis_index("c")
    s = lax.axis_index("s")
    wid = s * 2 + c

    def zbody(i, carry):
        for q in range(8):
            rows_v[i, pl.ds(q * 16, 16)] = jnp.zeros((16,), jnp.float32)
        return carry

    lax.fori_loop(0, 128, zbody, 0)
    for r in range(5):
        pltpu.sync_copy(rows_v.at[pl.ds(0, 128)],
                        acc_s.at[pl.ds(s * 640 + r * 128, 128)])
    plsc.subcore_barrier()

    base = wid * NCHUNK

    def start_idx_group(g, half):
        pltpu.async_copy(src_hbm.at[pl.ds(base + g * 8, 8)],
                         src_v.at[pl.ds(half, 8)], isem)
        pltpu.async_copy(dst_hbm.at[pl.ds(base + g * 8, 8)],
                         dst_v.at[pl.ds(half, 8)], isem)
        pltpu.async_copy(w_hbm.at[pl.ds(base + g * 8, 8)],
                         w_v.at[pl.ds(half, 8)], isem)

    def wait_idx_group(half):
        pltpu.make_async_copy(src_hbm.at[pl.ds(0, 8)],
                              src_v.at[pl.ds(half, 8)], isem).wait()
        pltpu.make_async_copy(dst_hbm.at[pl.ds(0, 8)],
                              dst_v.at[pl.ds(half, 8)], isem).wait()
        pltpu.make_async_copy(w_hbm.at[pl.ds(0, 8)],
                              w_v.at[pl.ds(half, 8)], isem).wait()

    start_idx_group(0, 0)
    wait_idx_group(0)
    start_idx_group(1, 8)
    pltpu.async_copy(xs_hbm.at[src_v.at[0]], rows_v.at[pl.ds(0, 128)], gsem)

    def loop(j, carry):
        b = lax.rem(j, 2) * 128
        jdiv = lax.div(j, 8)
        sub = lax.rem(j, 8)
        sel = lax.rem(jdiv, 2) * 8
        pltpu.make_async_copy(xs_hbm.at[src_v.at[sel + sub]],
                              rows_v.at[pl.ds(b, 128)], gsem).wait()

        @pl.when(j >= 1)
        def _():
            pltpu.make_async_copy(rows_v.at[pl.ds(128 - b, 128)],
                                  acc_s.at[dst_v.at[sel + sub]], ssem).wait()

        @pl.when(jnp.logical_and(sub == 7, j + 1 < NCHUNK))
        def _():
            wait_idx_group(8 - sel)

        @pl.when(j + 1 < NCHUNK)
        def _():
            j1 = j + 1
            r1 = lax.rem(lax.div(j1, 8), 2) * 8 + lax.rem(j1, 8)
            pltpu.async_copy(xs_hbm.at[src_v.at[r1]],
                             rows_v.at[pl.ds(128 - b, 128)], gsem)

        def sbody(g, carry2):
            w16 = w_v[sel + sub, pl.ds(g * 16, 16)]
            for t in range(16):
                wb = w16.at[jnp.full((16,), t, jnp.int32)].get(
                    mode="promise_in_bounds", unique_indices=False)
                row = b + g * 16 + t
                for q in range(8):
                    sl = pl.ds(q * 16, 16)
                    rows_v[row, sl] = rows_v[row, sl] * wb
            return carry2

        lax.fori_loop(0, 8, sbody, 0)
        pltpu.async_copy(rows_v.at[pl.ds(b, 128)],
                         acc_s.at[dst_v.at[sel + sub]], ssem, add=True)

        @pl.when(jnp.logical_and(sub == 7, jdiv + 2 < NGRP))
        def _():
            start_idx_group(jdiv + 2, sel)

        return carry

    lax.fori_loop(0, NCHUNK, loop, 0)
    pltpu.make_async_copy(rows_v.at[pl.ds(128, 128)],
                          acc_s.at[dst_v.at[15]], ssem).wait()
    plsc.subcore_barrier()
    pltpu.sync_copy(acc_s.at[pl.ds(s * 640, 640)],
                    out_hbm.at[pl.ds(c * NP + s * 640, 640)])


def _message_pass(src2d, dst2d, w2d, xs):
    return pl.kernel(
        _mp_body,
        out_type=jax.ShapeDtypeStruct((2 * NP, D), jnp.float32),
        mesh=plsc.VectorSubcoreMesh(core_axis_name="c", subcore_axis_name="s", num_cores=2, num_subcores=16),
        scratch_types=[
            pltpu.VMEM((16, 128), jnp.int32),
            pltpu.VMEM((16, 128), jnp.int32),
            pltpu.VMEM((16, 128), jnp.float32),
            pltpu.VMEM((256, D), jnp.float32),
            pltpu.VMEM_SHARED((NP, D), jnp.float32),
            pltpu.SemaphoreType.DMA,
            pltpu.SemaphoreType.DMA,
            pltpu.SemaphoreType.DMA,
        ],
    )(src2d, dst2d, w2d, xs)


def _dinv_body(dg_ref, o_ref):
    o_ref[...] = lax.rsqrt(dg_ref[0] + dg_ref[1] + (1.0 + 1e-10))


def _xs_body(x_ref, di_ref, o_ref):
    o_ref[...] = x_ref[...] * di_ref[...]


def _out_body(a0_ref, a1_ref, xs_ref, di_ref, th_ref, o_ref):
    pre = (a0_ref[...] + a1_ref[...] + xs_ref[...]) * di_ref[...]
    o_ref[...] = jnp.dot(pre, th_ref[...], preferred_element_type=jnp.float32)


def kernel(data, edge_list, weight_list, theta):
    n = data.shape[0]
    e0 = edge_list[:, 0].astype(jnp.int32)
    e1 = edge_list[:, 1].astype(jnp.int32)
    e2 = 2 * edge_list.shape[0]
    pad = E2P - e2
    src = jnp.concatenate([e0, e1, jnp.zeros((pad,), jnp.int32)])
    dst = jnp.concatenate([e1, e0, jnp.zeros((pad,), jnp.int32)])
    w2 = jnp.concatenate([weight_list, weight_list,
                          jnp.zeros((pad,), jnp.float32)])
    src2d = src.reshape(IDX_ROWS, 128)
    dst2d = dst.reshape(IDX_ROWS, 128)
    w2d = w2.reshape(IDX_ROWS, 128)
    datap = jnp.pad(data, ((0, NP - n), (0, 0)))

    deg_parts = _degrees(dst2d, w2d)

    dinv2d = pl.pallas_call(
        _dinv_body,
        out_shape=jax.ShapeDtypeStruct((NP // 128, 128), jnp.float32),
    )(deg_parts.reshape(2, NP // 128, 128))
    dinv_col = dinv2d.reshape(NP, 1)

    blk = 1024
    grid = NP // blk
    xs = pl.pallas_call(
        _xs_body,
        grid=(grid,),
        in_specs=[
            pl.BlockSpec((blk, D), lambda i: (i, 0)),
            pl.BlockSpec((blk, 1), lambda i: (i, 0)),
        ],
        out_specs=pl.BlockSpec((blk, D), lambda i: (i, 0)),
        out_shape=jax.ShapeDtypeStruct((NP, D), jnp.float32),
    )(datap, dinv_col)

    acc_parts = _message_pass(src2d, dst2d, w2d, xs)

    out = pl.pallas_call(
        _out_body,
        grid=(grid,),
        in_specs=[
            pl.BlockSpec((blk, D), lambda i: (i, 0)),
            pl.BlockSpec((blk, D), lambda i: (i, 0)),
            pl.BlockSpec((blk, D), lambda i: (i, 0)),
            pl.BlockSpec((blk, 1), lambda i: (i, 0)),
            pl.BlockSpec((D, D), lambda i: (0, 0)),
        ],
        out_specs=pl.BlockSpec((blk, D), lambda i: (i, 0)),
        out_shape=jax.ShapeDtypeStruct((NP, D), jnp.float32),
    )(acc_parts[:NP], acc_parts[NP:], xs, dinv_col, theta)
    return out[:n]


# speedup vs baseline: 1.1360x; 1.1360x over previous
import jax
import jax.numpy as jnp
from jax import lax
from jax.experimental import pallas as pl
from jax.experimental.pallas import tpu as pltpu
from jax.experimental.pallas import tpu_sc as plsc

NP = 10240
D = 128
NW = 32
NCHUNK = 160
EPT = NCHUNK * 128
E2P = NW * EPT
IDX_ROWS = E2P // 128


def _deg_body(dst_hbm, w_hbm, out_hbm, dst_v, w_v, deg_s, zb):
    c = lax.axis_index("c")
    s = lax.axis_index("s")
    wid = s * 2 + c
    for q in range(640 // 16):
        zb[pl.ds(q * 16, 16)] = jnp.zeros((16,), jnp.float32)
    pltpu.sync_copy(zb, deg_s.at[pl.ds(s * 640, 640)])
    plsc.subcore_barrier()
    pltpu.sync_copy(dst_hbm.at[pl.ds(wid * NCHUNK, NCHUNK)], dst_v)
    pltpu.sync_copy(w_hbm.at[pl.ds(wid * NCHUNK, NCHUNK)], w_v)

    def body(j, carry):
        pltpu.sync_copy(w_v.at[j], deg_s.at[dst_v.at[j]], add=True)
        return carry

    lax.fori_loop(0, NCHUNK, body, 0)
    plsc.subcore_barrier()
    pltpu.sync_copy(deg_s.at[pl.ds(s * 640, 640)],
                    out_hbm.at[pl.ds(c * NP + s * 640, 640)])


def _degrees(dst2d, w2d):
    return pl.kernel(
        _deg_body,
        out_type=jax.ShapeDtypeStruct((2 * NP,), jnp.float32),
        mesh=plsc.VectorSubcoreMesh(core_axis_name="c", subcore_axis_name="s", num_cores=2, num_subcores=16),
        scratch_types=[
            pltpu.VMEM((NCHUNK, 128), jnp.int32),
            pltpu.VMEM((NCHUNK, 128), jnp.float32),
            pltpu.VMEM_SHARED((NP,), jnp.float32),
            pltpu.VMEM((640,), jnp.float32),
        ],
    )(dst2d, w2d)


NGRP = NCHUNK // 8


def _mp_body(src_hbm, dst_hbm, w_hbm, xs_hbm, out_hbm,
             src_v, dst_v, w_v, rows_v, acc_s, gsem, isem, ssem):
    c = lax.axis_index("c")
    s = lax.axis_index("s")
    wid = s * 2 + c

    def zbody(i, carry):
        for q in range(8):
            rows_v[i, pl.ds(q * 16, 16)] = jnp.zeros((16,), jnp.float32)
        return carry

    lax.fori_loop(0, 128, zbody, 0)
    for r in range(5):
        pltpu.sync_copy(rows_v.at[pl.ds(0, 128)],
                        acc_s.at[pl.ds(s * 640 + r * 128, 128)])
    plsc.subcore_barrier()

    base = wid * NCHUNK

    def start_idx_group(g, half):
        pltpu.async_copy(src_hbm.at[pl.ds(base + g * 8, 8)],
                         src_v.at[pl.ds(half, 8)], isem)
        pltpu.async_copy(dst_hbm.at[pl.ds(base + g * 8, 8)],
                         dst_v.at[pl.ds(half, 8)], isem)
        pltpu.async_copy(w_hbm.at[pl.ds(base + g * 8, 8)],
                         w_v.at[pl.ds(half, 8)], isem)

    def wait_idx_group(half):
        pltpu.make_async_copy(src_hbm.at[pl.ds(0, 8)],
                              src_v.at[pl.ds(half, 8)], isem).wait()
        pltpu.make_async_copy(dst_hbm.at[pl.ds(0, 8)],
                              dst_v.at[pl.ds(half, 8)], isem).wait()
        pltpu.make_async_copy(w_hbm.at[pl.ds(0, 8)],
                              w_v.at[pl.ds(half, 8)], isem).wait()

    start_idx_group(0, 0)
    wait_idx_group(0)
    start_idx_group(1, 8)
    pltpu.async_copy(xs_hbm.at[src_v.at[0]], rows_v.at[pl.ds(0, 128)], gsem)

    def loop(j, carry):
        b = lax.rem(j, 2) * 128
        jdiv = lax.div(j, 8)
        sub = lax.rem(j, 8)
        sel = lax.rem(jdiv, 2) * 8
        pltpu.make_async_copy(xs_hbm.at[src_v.at[sel + sub]],
                              rows_v.at[pl.ds(b, 128)], gsem).wait()

        @pl.when(j >= 1)
        def _():
            pltpu.make_async_copy(rows_v.at[pl.ds(128 - b, 128)],
                                  acc_s.at[dst_v.at[sel + sub]], ssem).wait()

        @pl.when(jnp.logical_and(sub == 7, j + 1 < NCHUNK))
        def _():
            wait_idx_group(8 - sel)

        @pl.when(j + 1 < NCHUNK)
        def _():
            j1 = j + 1
            r1 = lax.rem(lax.div(j1, 8), 2) * 8 + lax.rem(j1, 8)
            pltpu.async_copy(xs_hbm.at[src_v.at[r1]],
                             rows_v.at[pl.ds(128 - b, 128)], gsem)

        @plsc.parallel_loop(0, 8, unroll=2)
        def sbody(g):
            w16 = w_v[sel + sub, pl.ds(g * 16, 16)]
            for t in range(16):
                wb = w16.at[jnp.full((16,), t, jnp.int32)].get(
                    mode="promise_in_bounds", unique_indices=False)
                row = b + g * 16 + t
                for q in range(8):
                    sl = pl.ds(q * 16, 16)
                    rows_v[row, sl] = rows_v[row, sl] * wb
        pltpu.async_copy(rows_v.at[pl.ds(b, 128)],
                         acc_s.at[dst_v.at[sel + sub]], ssem, add=True)

        @pl.when(jnp.logical_and(sub == 7, jdiv + 2 < NGRP))
        def _():
            start_idx_group(jdiv + 2, sel)

        return carry

    lax.fori_loop(0, NCHUNK, loop, 0)
    pltpu.make_async_copy(rows_v.at[pl.ds(128, 128)],
                          acc_s.at[dst_v.at[15]], ssem).wait()
    plsc.subcore_barrier()
    pltpu.sync_copy(acc_s.at[pl.ds(s * 640, 640)],
                    out_hbm.at[pl.ds(c * NP + s * 640, 640)])


def _message_pass(src2d, dst2d, w2d, xs):
    return pl.kernel(
        _mp_body,
        out_type=jax.ShapeDtypeStruct((2 * NP, D), jnp.float32),
        mesh=plsc.VectorSubcoreMesh(core_axis_name="c", subcore_axis_name="s", num_cores=2, num_subcores=16),
        scratch_types=[
            pltpu.VMEM((16, 128), jnp.int32),
            pltpu.VMEM((16, 128), jnp.int32),
            pltpu.VMEM((16, 128), jnp.float32),
            pltpu.VMEM((256, D), jnp.float32),
            pltpu.VMEM_SHARED((NP, D), jnp.float32),
            pltpu.SemaphoreType.DMA,
            pltpu.SemaphoreType.DMA,
            pltpu.SemaphoreType.DMA,
        ],
    )(src2d, dst2d, w2d, xs)


def _dinv_body(dg_ref, o_ref):
    o_ref[...] = lax.rsqrt(dg_ref[0] + dg_ref[1] + (1.0 + 1e-10))


def _xs_body(x_ref, di_ref, o_ref):
    o_ref[...] = x_ref[...] * di_ref[...]


def _out_body(a0_ref, a1_ref, xs_ref, di_ref, th_ref, o_ref):
    pre = (a0_ref[...] + a1_ref[...] + xs_ref[...]) * di_ref[...]
    o_ref[...] = jnp.dot(pre, th_ref[...], preferred_element_type=jnp.float32)


def kernel(data, edge_list, weight_list, theta):
    n = data.shape[0]
    e0 = edge_list[:, 0].astype(jnp.int32)
    e1 = edge_list[:, 1].astype(jnp.int32)
    e2 = 2 * edge_list.shape[0]
    pad = E2P - e2
    src = jnp.concatenate([e0, e1, jnp.zeros((pad,), jnp.int32)])
    dst = jnp.concatenate([e1, e0, jnp.zeros((pad,), jnp.int32)])
    w2 = jnp.concatenate([weight_list, weight_list,
                          jnp.zeros((pad,), jnp.float32)])
    src2d = src.reshape(IDX_ROWS, 128)
    dst2d = dst.reshape(IDX_ROWS, 128)
    w2d = w2.reshape(IDX_ROWS, 128)
    datap = jnp.pad(data, ((0, NP - n), (0, 0)))

    deg_parts = _degrees(dst2d, w2d)

    dinv2d = pl.pallas_call(
        _dinv_body,
        out_shape=jax.ShapeDtypeStruct((NP // 128, 128), jnp.float32),
    )(deg_parts.reshape(2, NP // 128, 128))
    dinv_col = dinv2d.reshape(NP, 1)

    blk = 1024
    grid = NP // blk
    xs = pl.pallas_call(
        _xs_body,
        grid=(grid,),
        in_specs=[
            pl.BlockSpec((blk, D), lambda i: (i, 0)),
            pl.BlockSpec((blk, 1), lambda i: (i, 0)),
        ],
        out_specs=pl.BlockSpec((blk, D), lambda i: (i, 0)),
        out_shape=jax.ShapeDtypeStruct((NP, D), jnp.float32),
    )(datap, dinv_col)

    acc_parts = _message_pass(src2d, dst2d, w2d, xs)

    out = pl.pallas_call(
        _out_body,
        grid=(grid,),
        in_specs=[
            pl.BlockSpec((blk, D), lambda i: (i, 0)),
            pl.BlockSpec((blk, D), lambda i: (i, 0)),
            pl.BlockSpec((blk, D), lambda i: (i, 0)),
            pl.BlockSpec((blk, 1), lambda i: (i, 0)),
            pl.BlockSpec((D, D), lambda i: (0, 0)),
        ],
        out_specs=pl.BlockSpec((blk, D), lambda i: (i, 0)),
        out_shape=jax.ShapeDtypeStruct((NP, D), jnp.float32),
    )(acc_parts[:NP], acc_parts[NP:], xs, dinv_col, theta)
    return out[:n]


# speedup vs baseline: 3.0049x; 2.6450x over previous
import jax
import jax.numpy as jnp
from jax import lax
from jax.experimental import pallas as pl
from jax.experimental.pallas import tpu as pltpu
from jax.experimental.pallas import tpu_sc as plsc

NP = 10240
D = 128
NW = 32
NCHUNK = 160
EPT = NCHUNK * 128
E2P = NW * EPT
IDX_ROWS = E2P // 128


def _deg_body(dst_hbm, w_hbm, out_hbm, dst_v, w_v, deg_s, zb):
    c = lax.axis_index("c")
    s = lax.axis_index("s")
    wid = s * 2 + c
    for q in range(640 // 16):
        zb[pl.ds(q * 16, 16)] = jnp.zeros((16,), jnp.float32)
    pltpu.sync_copy(zb, deg_s.at[pl.ds(s * 640, 640)])
    plsc.subcore_barrier()
    pltpu.sync_copy(dst_hbm.at[pl.ds(wid * NCHUNK, NCHUNK)], dst_v)
    pltpu.sync_copy(w_hbm.at[pl.ds(wid * NCHUNK, NCHUNK)], w_v)

    def body(j, carry):
        pltpu.sync_copy(w_v.at[j], deg_s.at[dst_v.at[j]], add=True)
        return carry

    lax.fori_loop(0, NCHUNK, body, 0)
    plsc.subcore_barrier()
    pltpu.sync_copy(deg_s.at[pl.ds(s * 640, 640)],
                    out_hbm.at[pl.ds(c * NP + s * 640, 640)])


def _degrees(dst2d, w2d):
    return pl.kernel(
        _deg_body,
        out_type=jax.ShapeDtypeStruct((2 * NP,), jnp.float32),
        mesh=plsc.VectorSubcoreMesh(core_axis_name="c", subcore_axis_name="s", num_cores=2, num_subcores=16),
        scratch_types=[
            pltpu.VMEM((NCHUNK, 128), jnp.int32),
            pltpu.VMEM((NCHUNK, 128), jnp.float32),
            pltpu.VMEM_SHARED((NP,), jnp.float32),
            pltpu.VMEM((640,), jnp.float32),
        ],
    )(dst2d, w2d)


NGRP = NCHUNK // 8


def _mp_body(src_hbm, dst_hbm, w_hbm, xs_hbm, out_hbm,
             src_v, dst_v, w_v, rows_v, acc_s, gsem, isem, ssem):
    c = lax.axis_index("c")
    s = lax.axis_index("s")
    wid = s * 2 + c

    def zbody(i, carry):
        for q in range(8):
            rows_v[i, pl.ds(q * 16, 16)] = jnp.zeros((16,), jnp.float32)
        return carry

    lax.fori_loop(0, 128, zbody, 0)
    for r in range(5):
        pltpu.sync_copy(rows_v.at[pl.ds(0, 128)],
                        acc_s.at[pl.ds(s * 640 + r * 128, 128)])
    plsc.subcore_barrier()

    base = wid * NCHUNK

    def start_idx_group(g, half):
        pltpu.async_copy(src_hbm.at[pl.ds(base + g * 8, 8)],
                         src_v.at[pl.ds(half, 8)], isem)
        pltpu.async_copy(dst_hbm.at[pl.ds(base + g * 8, 8)],
                         dst_v.at[pl.ds(half, 8)], isem)
        pltpu.async_copy(w_hbm.at[pl.ds(base + g * 8, 8)],
                         w_v.at[pl.ds(half, 8)], isem)

    def wait_idx_group(half):
        pltpu.make_async_copy(src_hbm.at[pl.ds(0, 8)],
                              src_v.at[pl.ds(half, 8)], isem).wait()
        pltpu.make_async_copy(dst_hbm.at[pl.ds(0, 8)],
                              dst_v.at[pl.ds(half, 8)], isem).wait()
        pltpu.make_async_copy(w_hbm.at[pl.ds(0, 8)],
                              w_v.at[pl.ds(half, 8)], isem).wait()

    start_idx_group(0, 0)
    wait_idx_group(0)
    start_idx_group(1, 8)
    pltpu.async_copy(xs_hbm.at[src_v.at[0]], rows_v.at[pl.ds(0, 128)], gsem)

    def loop(j, carry):
        b = lax.rem(j, 2) * 128
        jdiv = lax.div(j, 8)
        sub = lax.rem(j, 8)
        sel = lax.rem(jdiv, 2) * 8
        pltpu.make_async_copy(xs_hbm.at[src_v.at[sel + sub]],
                              rows_v.at[pl.ds(b, 128)], gsem).wait()

        @pl.when(j >= 1)
        def _():
            pltpu.make_async_copy(rows_v.at[pl.ds(128 - b, 128)],
                                  acc_s.at[dst_v.at[sel + sub]], ssem).wait()

        @pl.when(jnp.logical_and(sub == 7, j + 1 < NCHUNK))
        def _():
            wait_idx_group(8 - sel)

        @pl.when(j + 1 < NCHUNK)
        def _():
            j1 = j + 1
            r1 = lax.rem(lax.div(j1, 8), 2) * 8 + lax.rem(j1, 8)
            pltpu.async_copy(xs_hbm.at[src_v.at[r1]],
                             rows_v.at[pl.ds(128 - b, 128)], gsem)

        @plsc.parallel_loop(0, 8, unroll=2)
        def sbody(g):
            w16 = w_v[sel + sub, pl.ds(g * 16, 16)]
            for t in range(16):
                wb = w16.at[jnp.full((16,), t, jnp.int32)].get(
                    mode="promise_in_bounds", unique_indices=False)
                row = b + g * 16 + t
                for q in range(8):
                    sl = pl.ds(q * 16, 16)
                    rows_v[row, sl] = rows_v[row, sl] * wb
        pltpu.async_copy(rows_v.at[pl.ds(b, 128)],
                         acc_s.at[dst_v.at[sel + sub]], ssem, add=True)

        @pl.when(jnp.logical_and(sub == 7, jdiv + 2 < NGRP))
        def _():
            start_idx_group(jdiv + 2, sel)

        return carry

    lax.fori_loop(0, NCHUNK, loop, 0)
    pltpu.make_async_copy(rows_v.at[pl.ds(128, 128)],
                          acc_s.at[dst_v.at[15]], ssem).wait()
    plsc.subcore_barrier()
    pltpu.sync_copy(acc_s.at[pl.ds(s * 640, 640)],
                    out_hbm.at[pl.ds(c * NP + s * 640, 640)])


def _message_pass(src2d, dst2d, w2d, xs):
    return pl.kernel(
        _mp_body,
        out_type=jax.ShapeDtypeStruct((2 * NP, D), jnp.float32),
        mesh=plsc.VectorSubcoreMesh(core_axis_name="c", subcore_axis_name="s", num_cores=2, num_subcores=16),
        scratch_types=[
            pltpu.VMEM((16, 128), jnp.int32),
            pltpu.VMEM((16, 128), jnp.int32),
            pltpu.VMEM((16, 128), jnp.float32),
            pltpu.VMEM((256, D), jnp.float32),
            pltpu.VMEM_SHARED((NP, D), jnp.float32),
            pltpu.SemaphoreType.DMA,
            pltpu.SemaphoreType.DMA,
            pltpu.SemaphoreType.DMA,
        ],
    )(src2d, dst2d, w2d, xs)


def _dinv_body(dg_ref, o_ref):
    o_ref[...] = lax.rsqrt(dg_ref[0] + dg_ref[1] + (1.0 + 1e-10))


def _xs_body(x_ref, di_ref, o_ref):
    o_ref[...] = x_ref[...] * di_ref[...]


def _out_body(a0_ref, a1_ref, xs_ref, di_ref, th_ref, o_ref):
    pre = (a0_ref[...] + a1_ref[...] + xs_ref[...]) * di_ref[...]
    o_ref[...] = jnp.dot(pre, th_ref[...], preferred_element_type=jnp.float32)


def kernel(data, edge_list, weight_list, theta):
    n = data.shape[0]
    e0 = edge_list[:, 0].astype(jnp.int32)
    e1 = edge_list[:, 1].astype(jnp.int32)
    e2 = 2 * edge_list.shape[0]
    pad = E2P - e2
    pad_idx = jnp.arange(pad, dtype=jnp.int32) % n
    src = jnp.concatenate([e0, e1, pad_idx])
    dst = jnp.concatenate([e1, e0, pad_idx])
    w2 = jnp.concatenate([weight_list, weight_list,
                          jnp.zeros((pad,), jnp.float32)])
    src2d = src.reshape(IDX_ROWS, 128)
    dst2d = dst.reshape(IDX_ROWS, 128)
    w2d = w2.reshape(IDX_ROWS, 128)
    datap = jnp.pad(data, ((0, NP - n), (0, 0)))

    deg_parts = _degrees(dst2d, w2d)

    dinv2d = pl.pallas_call(
        _dinv_body,
        out_shape=jax.ShapeDtypeStruct((NP // 128, 128), jnp.float32),
    )(deg_parts.reshape(2, NP // 128, 128))
    dinv_col = dinv2d.reshape(NP, 1)

    blk = 1024
    grid = NP // blk
    xs = pl.pallas_call(
        _xs_body,
        grid=(grid,),
        in_specs=[
            pl.BlockSpec((blk, D), lambda i: (i, 0)),
            pl.BlockSpec((blk, 1), lambda i: (i, 0)),
        ],
        out_specs=pl.BlockSpec((blk, D), lambda i: (i, 0)),
        out_shape=jax.ShapeDtypeStruct((NP, D), jnp.float32),
    )(datap, dinv_col)

    acc_parts = _message_pass(src2d, dst2d, w2d, xs)

    out = pl.pallas_call(
        _out_body,
        grid=(grid,),
        in_specs=[
            pl.BlockSpec((blk, D), lambda i: (i, 0)),
            pl.BlockSpec((blk, D), lambda i: (i, 0)),
            pl.BlockSpec((blk, D), lambda i: (i, 0)),
            pl.BlockSpec((blk, 1), lambda i: (i, 0)),
            pl.BlockSpec((D, D), lambda i: (0, 0)),
        ],
        out_specs=pl.BlockSpec((blk, D), lambda i: (i, 0)),
        out_shape=jax.ShapeDtypeStruct((NP, D), jnp.float32),
    )(acc_parts[:NP], acc_parts[NP:], xs, dinv_col, theta)
    return out[:n]


# speedup vs baseline: 3.1541x; 1.0496x over previous
import jax
import jax.numpy as jnp
from jax import lax
from jax.experimental import pallas as pl
from jax.experimental.pallas import tpu as pltpu
from jax.experimental.pallas import tpu_sc as plsc

NP = 10240
D = 128
NW = 32
NCHUNK = 160
EPT = NCHUNK * 128
E2P = NW * EPT
IDX_ROWS = E2P // 128


def _deg_body(dst_hbm, w_hbm, out_hbm, dst_v, w_v, deg_s, zb):
    c = lax.axis_index("c")
    s = lax.axis_index("s")
    wid = s * 2 + c
    for q in range(640 // 16):
        zb[pl.ds(q * 16, 16)] = jnp.zeros((16,), jnp.float32)
    pltpu.sync_copy(zb, deg_s.at[pl.ds(s * 640, 640)])
    plsc.subcore_barrier()
    pltpu.sync_copy(dst_hbm.at[pl.ds(wid * NCHUNK, NCHUNK)], dst_v)
    pltpu.sync_copy(w_hbm.at[pl.ds(wid * NCHUNK, NCHUNK)], w_v)

    def body(j, carry):
        pltpu.sync_copy(w_v.at[j], deg_s.at[dst_v.at[j]], add=True)
        return carry

    lax.fori_loop(0, NCHUNK, body, 0)
    plsc.subcore_barrier()
    pltpu.sync_copy(deg_s.at[pl.ds(s * 640, 640)],
                    out_hbm.at[pl.ds(c * NP + s * 640, 640)])


def _degrees(dst2d, w2d):
    return pl.kernel(
        _deg_body,
        out_type=jax.ShapeDtypeStruct((2 * NP,), jnp.float32),
        mesh=plsc.VectorSubcoreMesh(core_axis_name="c", subcore_axis_name="s", num_cores=2, num_subcores=16),
        scratch_types=[
            pltpu.VMEM((NCHUNK, 128), jnp.int32),
            pltpu.VMEM((NCHUNK, 128), jnp.float32),
            pltpu.VMEM_SHARED((NP,), jnp.float32),
            pltpu.VMEM((640,), jnp.float32),
        ],
    )(dst2d, w2d)


NACC = 10000
NGRP2 = NCHUNK // 2


def _mp_body(src_hbm, dst_hbm, w_hbm, xs_hbm, out_hbm,
             src_v, dst_v, w_v, rows_v, acc_s, gsem, isem, ssem):
    c = lax.axis_index("c")
    s = lax.axis_index("s")
    wid = s * 2 + c

    def zbody(i, carry):
        for q in range(8):
            rows_v[i, pl.ds(q * 16, 16)] = jnp.zeros((16,), jnp.float32)
        return carry

    lax.fori_loop(0, 128, zbody, 0)
    for r in range(4):
        pltpu.sync_copy(rows_v.at[pl.ds(0, 128)],
                        acc_s.at[pl.ds(s * 632 + r * 128, 128)])

    @pl.when(s < 15)
    def _():
        pltpu.sync_copy(rows_v.at[pl.ds(0, 120)],
                        acc_s.at[pl.ds(s * 632 + 512, 120)])

    @pl.when(s == 15)
    def _():
        pltpu.sync_copy(rows_v.at[pl.ds(0, 8)],
                        acc_s.at[pl.ds(s * 632 + 512, 8)])

    plsc.subcore_barrier()

    base = wid * NCHUNK

    def start_idx_group(g, half):
        pltpu.async_copy(src_hbm.at[pl.ds(base + g * 2, 2)],
                         src_v.at[pl.ds(half, 2)], isem)
        pltpu.async_copy(dst_hbm.at[pl.ds(base + g * 2, 2)],
                         dst_v.at[pl.ds(half, 2)], isem)
        pltpu.async_copy(w_hbm.at[pl.ds(base + g * 2, 2)],
                         w_v.at[pl.ds(half, 2)], isem)

    def wait_idx_group(half):
        pltpu.make_async_copy(src_hbm.at[pl.ds(0, 2)],
                              src_v.at[pl.ds(half, 2)], isem).wait()
        pltpu.make_async_copy(dst_hbm.at[pl.ds(0, 2)],
                              dst_v.at[pl.ds(half, 2)], isem).wait()
        pltpu.make_async_copy(w_hbm.at[pl.ds(0, 2)],
                              w_v.at[pl.ds(half, 2)], isem).wait()

    start_idx_group(0, 0)
    wait_idx_group(0)
    start_idx_group(1, 2)
    pltpu.async_copy(xs_hbm.at[src_v.at[0]], rows_v.at[pl.ds(0, 128)], gsem)
    pltpu.async_copy(xs_hbm.at[src_v.at[1]], rows_v.at[pl.ds(128, 128)], gsem)

    def loop(j, carry):
        b = lax.rem(j, 3) * 128
        jdiv = lax.div(j, 2)
        sub = lax.rem(j, 2)
        irow = lax.rem(jdiv, 2) * 2 + sub

        pltpu.make_async_copy(xs_hbm.at[src_v.at[irow]],
                              rows_v.at[pl.ds(b, 128)], gsem).wait()

        @pl.when(jnp.logical_and(sub == 0, jdiv + 1 < NGRP2))
        def _():
            wait_idx_group(lax.rem(jdiv + 1, 2) * 2)

        @plsc.parallel_loop(0, 8, unroll=2)
        def sbody(g):
            w16 = w_v[irow, pl.ds(g * 16, 16)]
            for t in range(16):
                wb = w16.at[jnp.full((16,), t, jnp.int32)].get(
                    mode="promise_in_bounds", unique_indices=False)
                row = b + g * 16 + t
                for q in range(8):
                    sl = pl.ds(q * 16, 16)
                    rows_v[row, sl] = rows_v[row, sl] * wb

        @pl.when(j >= 1)
        def _():
            pltpu.make_async_copy(
                rows_v.at[pl.ds(lax.rem(j + 2, 3) * 128, 128)],
                acc_s.at[dst_v.at[irow]], ssem).wait()

        pltpu.async_copy(rows_v.at[pl.ds(b, 128)],
                         acc_s.at[dst_v.at[irow]], ssem, add=True)

        @pl.when(j + 2 < NCHUNK)
        def _():
            j2 = j + 2
            irow2 = lax.rem(lax.div(j2, 2), 2) * 2 + lax.rem(j2, 2)
            pltpu.async_copy(xs_hbm.at[src_v.at[irow2]],
                             rows_v.at[pl.ds(lax.rem(j2, 3) * 128, 128)],
                             gsem)

        @pl.when(jnp.logical_and(sub == 1, jdiv + 2 < NGRP2))
        def _():
            start_idx_group(jdiv + 2, lax.rem(jdiv, 2) * 2)

        return carry

    lax.fori_loop(0, NCHUNK, loop, 0)
    pltpu.make_async_copy(
        rows_v.at[pl.ds(lax.rem(NCHUNK - 1, 3) * 128, 128)],
        acc_s.at[dst_v.at[3]], ssem).wait()
    plsc.subcore_barrier()
    for r in range(4):
        pltpu.sync_copy(
            acc_s.at[pl.ds(s * 632 + r * 128, 128)],
            out_hbm.at[pl.ds(c * NACC + s * 632 + r * 128, 128)])

    @pl.when(s < 15)
    def _():
        pltpu.sync_copy(acc_s.at[pl.ds(s * 632 + 512, 120)],
                        out_hbm.at[pl.ds(c * NACC + s * 632 + 512, 120)])

    @pl.when(s == 15)
    def _():
        pltpu.sync_copy(acc_s.at[pl.ds(s * 632 + 512, 8)],
                        out_hbm.at[pl.ds(c * NACC + s * 632 + 512, 8)])


def _message_pass(src2d, dst2d, w2d, xs):
    return pl.kernel(
        _mp_body,
        out_type=jax.ShapeDtypeStruct((2 * NACC, D), jnp.float32),
        mesh=plsc.VectorSubcoreMesh(core_axis_name="c", subcore_axis_name="s", num_cores=2, num_subcores=16),
        scratch_types=[
            pltpu.VMEM((4, 128), jnp.int32),
            pltpu.VMEM((4, 128), jnp.int32),
            pltpu.VMEM((4, 128), jnp.float32),
            pltpu.VMEM((384, D), jnp.float32),
            pltpu.VMEM_SHARED((NACC, D), jnp.float32),
            pltpu.SemaphoreType.DMA,
            pltpu.SemaphoreType.DMA,
            pltpu.SemaphoreType.DMA,
        ],
    )(src2d, dst2d, w2d, xs)


def _norm_body(dg0_ref, dg1_ref, x_ref, xs_ref, di_ref):
    di = lax.rsqrt(dg0_ref[...] + dg1_ref[...] + (1.0 + 1e-10))
    di_ref[...] = di
    xs_ref[...] = x_ref[...] * di


def _out_body(a0_ref, a1_ref, xs_ref, di_ref, th_ref, o_ref):
    pre = (a0_ref[...] + a1_ref[...] + xs_ref[...]) * di_ref[...]
    o_ref[...] = jnp.dot(pre, th_ref[...], preferred_element_type=jnp.float32)


def kernel(data, edge_list, weight_list, theta):
    n = data.shape[0]
    e0 = edge_list[:, 0].astype(jnp.int32)
    e1 = edge_list[:, 1].astype(jnp.int32)
    e2 = 2 * edge_list.shape[0]
    pad = E2P - e2
    pad_idx = jnp.arange(pad, dtype=jnp.int32) % n
    src = jnp.concatenate([e0, e1, pad_idx])
    dst = jnp.concatenate([e1, e0, pad_idx])
    w2 = jnp.concatenate([weight_list, weight_list,
                          jnp.zeros((pad,), jnp.float32)])
    src2d = src.reshape(IDX_ROWS, 128)
    dst2d = dst.reshape(IDX_ROWS, 128)
    w2d = w2.reshape(IDX_ROWS, 128)

    deg_parts = _degrees(dst2d, w2d)
    d0 = deg_parts[:NACC].reshape(NACC, 1)
    d1 = deg_parts[NP:NP + NACC].reshape(NACC, 1)

    blk = 400
    grid = NACC // blk
    xs, dinv_col = pl.pallas_call(
        _norm_body,
        grid=(grid,),
        in_specs=[
            pl.BlockSpec((blk, 1), lambda i: (i, 0)),
            pl.BlockSpec((blk, 1), lambda i: (i, 0)),
            pl.BlockSpec((blk, D), lambda i: (i, 0)),
        ],
        out_specs=[
            pl.BlockSpec((blk, D), lambda i: (i, 0)),
            pl.BlockSpec((blk, 1), lambda i: (i, 0)),
        ],
        out_shape=[
            jax.ShapeDtypeStruct((NACC, D), jnp.float32),
            jax.ShapeDtypeStruct((NACC, 1), jnp.float32),
        ],
    )(d0, d1, data)

    acc_parts = _message_pass(src2d, dst2d, w2d, xs)

    out = pl.pallas_call(
        _out_body,
        grid=(grid,),
        in_specs=[
            pl.BlockSpec((blk, D), lambda i: (i, 0)),
            pl.BlockSpec((blk, D), lambda i: (i, 0)),
            pl.BlockSpec((blk, D), lambda i: (i, 0)),
            pl.BlockSpec((blk, 1), lambda i: (i, 0)),
            pl.BlockSpec((D, D), lambda i: (0, 0)),
        ],
        out_specs=pl.BlockSpec((blk, D), lambda i: (i, 0)),
        out_shape=jax.ShapeDtypeStruct((NACC, D), jnp.float32),
    )(acc_parts[:NACC], acc_parts[NACC:], xs, dinv_col, theta)
    return out[:n]


# speedup vs baseline: 3.2542x; 1.0318x over previous
import jax
import jax.numpy as jnp
from jax import lax
from jax.experimental import pallas as pl
from jax.experimental.pallas import tpu as pltpu
from jax.experimental.pallas import tpu_sc as plsc

NP = 10240
D = 128
NW = 32
NCHUNK = 160
EPT = NCHUNK * 128
E2P = NW * EPT
IDX_ROWS = E2P // 128


def _deg_body(dst_hbm, w_hbm, out_hbm, dst_v, w_v, deg_s, zb, dsem):
    c = lax.axis_index("c")
    s = lax.axis_index("s")
    wid = s * 2 + c
    for q in range(640 // 16):
        zb[pl.ds(q * 16, 16)] = jnp.zeros((16,), jnp.float32)
    pltpu.sync_copy(zb, deg_s.at[pl.ds(s * 640, 640)])
    plsc.subcore_barrier()
    pltpu.sync_copy(dst_hbm.at[pl.ds(wid * NCHUNK, NCHUNK)], dst_v)
    pltpu.sync_copy(w_hbm.at[pl.ds(wid * NCHUNK, NCHUNK)], w_v)

    def body(j, carry):
        pltpu.async_copy(w_v.at[j], deg_s.at[dst_v.at[j]], dsem, add=True)
        @pl.when(j >= 8)
        def _():
            pltpu.make_async_copy(w_v.at[j], deg_s.at[dst_v.at[j]],
                                  dsem).wait()
        return carry

    lax.fori_loop(0, NCHUNK, body, 0)

    def drain(j, carry):
        pltpu.make_async_copy(w_v.at[j], deg_s.at[dst_v.at[j]], dsem).wait()
        return carry

    lax.fori_loop(0, 8, drain, 0)
    plsc.subcore_barrier()
    pltpu.sync_copy(deg_s.at[pl.ds(s * 640, 640)],
                    out_hbm.at[pl.ds(c * NP + s * 640, 640)])


def _degrees(dst2d, w2d):
    return pl.kernel(
        _deg_body,
        out_type=jax.ShapeDtypeStruct((2 * NP,), jnp.float32),
        mesh=plsc.VectorSubcoreMesh(core_axis_name="c", subcore_axis_name="s", num_cores=2, num_subcores=16),
        scratch_types=[
            pltpu.VMEM((NCHUNK, 128), jnp.int32),
            pltpu.VMEM((NCHUNK, 128), jnp.float32),
            pltpu.VMEM_SHARED((NP,), jnp.float32),
            pltpu.VMEM((640,), jnp.float32),
            pltpu.SemaphoreType.DMA,
        ],
    )(dst2d, w2d)


NACC = 10000
NGRP2 = NCHUNK // 2


def _mp_body(src_hbm, dst_hbm, w_hbm, xs_hbm, out_hbm,
             src_v, dst_v, w_v, rows_v, acc_s, gsem, isem, ssem):
    c = lax.axis_index("c")
    s = lax.axis_index("s")
    wid = s * 2 + c

    def zbody(i, carry):
        for q in range(8):
            rows_v[i, pl.ds(q * 16, 16)] = jnp.zeros((16,), jnp.float32)
        return carry

    lax.fori_loop(0, 128, zbody, 0)
    for r in range(4):
        pltpu.sync_copy(rows_v.at[pl.ds(0, 128)],
                        acc_s.at[pl.ds(s * 632 + r * 128, 128)])

    @pl.when(s < 15)
    def _():
        pltpu.sync_copy(rows_v.at[pl.ds(0, 120)],
                        acc_s.at[pl.ds(s * 632 + 512, 120)])

    @pl.when(s == 15)
    def _():
        pltpu.sync_copy(rows_v.at[pl.ds(0, 8)],
                        acc_s.at[pl.ds(s * 632 + 512, 8)])

    plsc.subcore_barrier()

    base = wid * NCHUNK

    def start_idx_group(g, half):
        pltpu.async_copy(src_hbm.at[pl.ds(base + g * 2, 2)],
                         src_v.at[pl.ds(half, 2)], isem)
        pltpu.async_copy(dst_hbm.at[pl.ds(base + g * 2, 2)],
                         dst_v.at[pl.ds(half, 2)], isem)
        pltpu.async_copy(w_hbm.at[pl.ds(base + g * 2, 2)],
                         w_v.at[pl.ds(half, 2)], isem)

    def wait_idx_group(half):
        pltpu.make_async_copy(src_hbm.at[pl.ds(0, 2)],
                              src_v.at[pl.ds(half, 2)], isem).wait()
        pltpu.make_async_copy(dst_hbm.at[pl.ds(0, 2)],
                              dst_v.at[pl.ds(half, 2)], isem).wait()
        pltpu.make_async_copy(w_hbm.at[pl.ds(0, 2)],
                              w_v.at[pl.ds(half, 2)], isem).wait()

    start_idx_group(0, 0)
    wait_idx_group(0)
    start_idx_group(1, 2)
    pltpu.async_copy(xs_hbm.at[src_v.at[0]], rows_v.at[pl.ds(0, 128)], gsem)
    pltpu.async_copy(xs_hbm.at[src_v.at[1]], rows_v.at[pl.ds(128, 128)], gsem)

    def loop(j, carry):
        b = lax.rem(j, 3) * 128
        jdiv = lax.div(j, 2)
        sub = lax.rem(j, 2)
        irow = lax.rem(jdiv, 2) * 2 + sub

        pltpu.make_async_copy(xs_hbm.at[src_v.at[irow]],
                              rows_v.at[pl.ds(b, 128)], gsem).wait()

        @pl.when(jnp.logical_and(sub == 0, jdiv + 1 < NGRP2))
        def _():
            wait_idx_group(lax.rem(jdiv + 1, 2) * 2)

        @plsc.parallel_loop(0, 8, unroll=4)
        def sbody(g):
            w16 = w_v[irow, pl.ds(g * 16, 16)]
            for t in range(16):
                wb = w16.at[jnp.full((16,), t, jnp.int32)].get(
                    mode="promise_in_bounds", unique_indices=False)
                row = b + g * 16 + t
                for q in range(8):
                    sl = pl.ds(q * 16, 16)
                    rows_v[row, sl] = rows_v[row, sl] * wb

        @pl.when(j >= 1)
        def _():
            pltpu.make_async_copy(
                rows_v.at[pl.ds(lax.rem(j + 2, 3) * 128, 128)],
                acc_s.at[dst_v.at[irow]], ssem).wait()

        pltpu.async_copy(rows_v.at[pl.ds(b, 128)],
                         acc_s.at[dst_v.at[irow]], ssem, add=True)

        @pl.when(j + 2 < NCHUNK)
        def _():
            j2 = j + 2
            irow2 = lax.rem(lax.div(j2, 2), 2) * 2 + lax.rem(j2, 2)
            pltpu.async_copy(xs_hbm.at[src_v.at[irow2]],
                             rows_v.at[pl.ds(lax.rem(j2, 3) * 128, 128)],
                             gsem)

        @pl.when(jnp.logical_and(sub == 1, jdiv + 2 < NGRP2))
        def _():
            start_idx_group(jdiv + 2, lax.rem(jdiv, 2) * 2)

        return carry

    lax.fori_loop(0, NCHUNK, loop, 0)
    pltpu.make_async_copy(
        rows_v.at[pl.ds(lax.rem(NCHUNK - 1, 3) * 128, 128)],
        acc_s.at[dst_v.at[3]], ssem).wait()
    plsc.subcore_barrier()
    for r in range(4):
        pltpu.sync_copy(
            acc_s.at[pl.ds(s * 632 + r * 128, 128)],
            out_hbm.at[pl.ds(c * NACC + s * 632 + r * 128, 128)])

    @pl.when(s < 15)
    def _():
        pltpu.sync_copy(acc_s.at[pl.ds(s * 632 + 512, 120)],
                        out_hbm.at[pl.ds(c * NACC + s * 632 + 512, 120)])

    @pl.when(s == 15)
    def _():
        pltpu.sync_copy(acc_s.at[pl.ds(s * 632 + 512, 8)],
                        out_hbm.at[pl.ds(c * NACC + s * 632 + 512, 8)])


def _message_pass(src2d, dst2d, w2d, xs):
    return pl.kernel(
        _mp_body,
        out_type=jax.ShapeDtypeStruct((2 * NACC, D), jnp.float32),
        mesh=plsc.VectorSubcoreMesh(core_axis_name="c", subcore_axis_name="s", num_cores=2, num_subcores=16),
        scratch_types=[
            pltpu.VMEM((4, 128), jnp.int32),
            pltpu.VMEM((4, 128), jnp.int32),
            pltpu.VMEM((4, 128), jnp.float32),
            pltpu.VMEM((384, D), jnp.float32),
            pltpu.VMEM_SHARED((NACC, D), jnp.float32),
            pltpu.SemaphoreType.DMA,
            pltpu.SemaphoreType.DMA,
            pltpu.SemaphoreType.DMA,
        ],
    )(src2d, dst2d, w2d, xs)


def _norm_body(dg0_ref, dg1_ref, x_ref, xs_ref, di_ref):
    di = lax.rsqrt(dg0_ref[...] + dg1_ref[...] + (1.0 + 1e-10))
    di_ref[...] = di
    xs_ref[...] = x_ref[...] * di


def _out_body(a0_ref, a1_ref, xs_ref, di_ref, th_ref, o_ref):
    pre = (a0_ref[...] + a1_ref[...] + xs_ref[...]) * di_ref[...]
    o_ref[...] = jnp.dot(pre, th_ref[...], preferred_element_type=jnp.float32)


def kernel(data, edge_list, weight_list, theta):
    n = data.shape[0]
    e0 = edge_list[:, 0].astype(jnp.int32)
    e1 = edge_list[:, 1].astype(jnp.int32)
    e2 = 2 * edge_list.shape[0]
    pad = E2P - e2
    pad_idx = jnp.arange(pad, dtype=jnp.int32) % n
    src = jnp.concatenate([e0, e1, pad_idx])
    dst = jnp.concatenate([e1, e0, pad_idx])
    w2 = jnp.concatenate([weight_list, weight_list,
                          jnp.zeros((pad,), jnp.float32)])
    src2d = src.reshape(IDX_ROWS, 128)
    dst2d = dst.reshape(IDX_ROWS, 128)
    w2d = w2.reshape(IDX_ROWS, 128)

    deg_parts = _degrees(dst2d, w2d)
    d0 = deg_parts[:NACC].reshape(NACC, 1)
    d1 = deg_parts[NP:NP + NACC].reshape(NACC, 1)

    blk = 400
    grid = NACC // blk
    xs, dinv_col = pl.pallas_call(
        _norm_body,
        grid=(grid,),
        in_specs=[
            pl.BlockSpec((blk, 1), lambda i: (i, 0)),
            pl.BlockSpec((blk, 1), lambda i: (i, 0)),
            pl.BlockSpec((blk, D), lambda i: (i, 0)),
        ],
        out_specs=[
            pl.BlockSpec((blk, D), lambda i: (i, 0)),
            pl.BlockSpec((blk, 1), lambda i: (i, 0)),
        ],
        out_shape=[
            jax.ShapeDtypeStruct((NACC, D), jnp.float32),
            jax.ShapeDtypeStruct((NACC, 1), jnp.float32),
        ],
    )(d0, d1, data)

    acc_parts = _message_pass(src2d, dst2d, w2d, xs)

    out = pl.pallas_call(
        _out_body,
        grid=(grid,),
        in_specs=[
            pl.BlockSpec((blk, D), lambda i: (i, 0)),
            pl.BlockSpec((blk, D), lambda i: (i, 0)),
            pl.BlockSpec((blk, D), lambda i: (i, 0)),
            pl.BlockSpec((blk, 1), lambda i: (i, 0)),
            pl.BlockSpec((D, D), lambda i: (0, 0)),
        ],
        out_specs=pl.BlockSpec((blk, D), lambda i: (i, 0)),
        out_shape=jax.ShapeDtypeStruct((NACC, D), jnp.float32),
    )(acc_parts[:NACC], acc_parts[NACC:], xs, dinv_col, theta)
    return out[:n]


# speedup vs baseline: 3.3271x; 1.0224x over previous
import jax
import jax.numpy as jnp
from jax import lax
from jax.experimental import pallas as pl
from jax.experimental.pallas import tpu as pltpu
from jax.experimental.pallas import tpu_sc as plsc

NP = 10240
D = 128
NW = 32
NCHUNK = 160
EPT = NCHUNK * 128
E2P = NW * EPT
IDX_ROWS = E2P // 128


def _deg_body(dst_hbm, w_hbm, out_hbm, dst_v, w_v, deg_s, zb, dsem):
    c = lax.axis_index("c")
    s = lax.axis_index("s")
    wid = s * 2 + c
    for q in range(640 // 16):
        zb[pl.ds(q * 16, 16)] = jnp.zeros((16,), jnp.float32)

    pltpu.sync_copy(zb, deg_s.at[pl.ds(s * 640, 640)])
    plsc.subcore_barrier()
    pltpu.sync_copy(dst_hbm.at[pl.ds(wid * NCHUNK, NCHUNK)], dst_v)
    pltpu.sync_copy(w_hbm.at[pl.ds(wid * NCHUNK, NCHUNK)], w_v)

    def body(j, carry):
        pltpu.async_copy(w_v.at[j], deg_s.at[dst_v.at[j]], dsem, add=True)
        @pl.when(j >= 8)
        def _():
            pltpu.make_async_copy(w_v.at[j], deg_s.at[dst_v.at[j]],
                                  dsem).wait()
        return carry

    lax.fori_loop(0, NCHUNK, body, 0)

    def drain(j, carry):
        pltpu.make_async_copy(w_v.at[j], deg_s.at[dst_v.at[j]], dsem).wait()
        return carry

    lax.fori_loop(0, 8, drain, 0)
    plsc.subcore_barrier()

    pltpu.sync_copy(deg_s.at[pl.ds(s * 640, 640)],
                    out_hbm.at[pl.ds(c * NP + s * 640, 640)])


def _degrees(dst2d, w2d):
    return pl.kernel(
        _deg_body,
        out_type=jax.ShapeDtypeStruct((2 * NP,), jnp.float32),
        mesh=plsc.VectorSubcoreMesh(core_axis_name="c", subcore_axis_name="s", num_cores=2, num_subcores=16),
        scratch_types=[
            pltpu.VMEM((NCHUNK, 128), jnp.int32),
            pltpu.VMEM((NCHUNK, 128), jnp.float32),
            pltpu.VMEM_SHARED((NP,), jnp.float32),
            pltpu.VMEM((640,), jnp.float32),
            pltpu.SemaphoreType.DMA,
        ],
    )(dst2d, w2d)


NACC = 10000
NGRP2 = NCHUNK // 2


def _mp_body(src_hbm, dst_hbm, w_hbm, xs_hbm, out_hbm,
             src_v, dst_v, w_v, rows_v, acc_s, gsem, isem, ssem):
    c = lax.axis_index("c")
    s = lax.axis_index("s")
    wid = s * 2 + c

    def zbody(i, carry):
        for q in range(8):
            rows_v[i, pl.ds(q * 16, 16)] = jnp.zeros((16,), jnp.float32)
        return carry

    lax.fori_loop(0, 128, zbody, 0)
    for r in range(4):
        pltpu.sync_copy(rows_v.at[pl.ds(0, 128)],
                        acc_s.at[pl.ds(s * 632 + r * 128, 128)])

    @pl.when(s < 15)
    def _():
        pltpu.sync_copy(rows_v.at[pl.ds(0, 120)],
                        acc_s.at[pl.ds(s * 632 + 512, 120)])

    @pl.when(s == 15)
    def _():
        pltpu.sync_copy(rows_v.at[pl.ds(0, 8)],
                        acc_s.at[pl.ds(s * 632 + 512, 8)])

    plsc.subcore_barrier()

    base = wid * NCHUNK

    def start_idx_group(g, half):
        pltpu.async_copy(src_hbm.at[pl.ds(base + g * 2, 2)],
                         src_v.at[pl.ds(half, 2)], isem)
        pltpu.async_copy(dst_hbm.at[pl.ds(base + g * 2, 2)],
                         dst_v.at[pl.ds(half, 2)], isem)
        pltpu.async_copy(w_hbm.at[pl.ds(base + g * 2, 2)],
                         w_v.at[pl.ds(half, 2)], isem)

    def wait_idx_group(half):
        pltpu.make_async_copy(src_hbm.at[pl.ds(0, 2)],
                              src_v.at[pl.ds(half, 2)], isem).wait()
        pltpu.make_async_copy(dst_hbm.at[pl.ds(0, 2)],
                              dst_v.at[pl.ds(half, 2)], isem).wait()
        pltpu.make_async_copy(w_hbm.at[pl.ds(0, 2)],
                              w_v.at[pl.ds(half, 2)], isem).wait()

    start_idx_group(0, 0)
    wait_idx_group(0)
    start_idx_group(1, 2)
    pltpu.async_copy(xs_hbm.at[src_v.at[0]], rows_v.at[pl.ds(0, 128)], gsem)
    pltpu.async_copy(xs_hbm.at[src_v.at[1]], rows_v.at[pl.ds(128, 128)], gsem)

    def loop(j, carry):
        b = lax.rem(j, 3) * 128
        jdiv = lax.div(j, 2)
        sub = lax.rem(j, 2)
        irow = lax.rem(jdiv, 2) * 2 + sub

        pltpu.make_async_copy(xs_hbm.at[src_v.at[irow]],
                              rows_v.at[pl.ds(b, 128)], gsem).wait()

        @pl.when(jnp.logical_and(sub == 0, jdiv + 1 < NGRP2))
        def _():
            wait_idx_group(lax.rem(jdiv + 1, 2) * 2)

        @plsc.parallel_loop(0, 8, unroll=4)
        def sbody(g):
            w16 = w_v[irow, pl.ds(g * 16, 16)]
            for t in range(16):
                wb = w16.at[jnp.full((16,), t, jnp.int32)].get(
                    mode="promise_in_bounds", unique_indices=False)
                row = b + g * 16 + t
                for q in range(8):
                    sl = pl.ds(q * 16, 16)
                    rows_v[row, sl] = rows_v[row, sl] * wb

        @pl.when(j >= 1)
        def _():
            pltpu.make_async_copy(
                rows_v.at[pl.ds(lax.rem(j + 2, 3) * 128, 128)],
                acc_s.at[dst_v.at[irow]], ssem).wait()

        pltpu.async_copy(rows_v.at[pl.ds(b, 128)],
                         acc_s.at[dst_v.at[irow]], ssem, add=True)

        @pl.when(j + 2 < NCHUNK)
        def _():
            j2 = j + 2
            irow2 = lax.rem(lax.div(j2, 2), 2) * 2 + lax.rem(j2, 2)
            pltpu.async_copy(xs_hbm.at[src_v.at[irow2]],
                             rows_v.at[pl.ds(lax.rem(j2, 3) * 128, 128)],
                             gsem)

        @pl.when(jnp.logical_and(sub == 1, jdiv + 2 < NGRP2))
        def _():
            start_idx_group(jdiv + 2, lax.rem(jdiv, 2) * 2)

        return carry

    lax.fori_loop(0, NCHUNK, loop, 0)
    pltpu.make_async_copy(
        rows_v.at[pl.ds(lax.rem(NCHUNK - 1, 3) * 128, 128)],
        acc_s.at[dst_v.at[3]], ssem).wait()
    plsc.subcore_barrier()
    for r in range(4):
        pltpu.sync_copy(
            acc_s.at[pl.ds(s * 632 + r * 128, 128)],
            out_hbm.at[pl.ds(c * NACC + s * 632 + r * 128, 128)])

    @pl.when(s < 15)
    def _():
        pltpu.sync_copy(acc_s.at[pl.ds(s * 632 + 512, 120)],
                        out_hbm.at[pl.ds(c * NACC + s * 632 + 512, 120)])

    @pl.when(s == 15)
    def _():
        pltpu.sync_copy(acc_s.at[pl.ds(s * 632 + 512, 8)],
                        out_hbm.at[pl.ds(c * NACC + s * 632 + 512, 8)])


def _message_pass(src2d, dst2d, w2d, xs):
    return pl.kernel(
        _mp_body,
        out_type=jax.ShapeDtypeStruct((2 * NACC, D), jnp.float32),
        mesh=plsc.VectorSubcoreMesh(core_axis_name="c", subcore_axis_name="s", num_cores=2, num_subcores=16),
        scratch_types=[
            pltpu.VMEM((4, 128), jnp.int32),
            pltpu.VMEM((4, 128), jnp.int32),
            pltpu.VMEM((4, 128), jnp.float32),
            pltpu.VMEM((384, D), jnp.float32),
            pltpu.VMEM_SHARED((NACC, D), jnp.float32),
            pltpu.SemaphoreType.DMA,
            pltpu.SemaphoreType.DMA,
            pltpu.SemaphoreType.DMA,
        ],
    )(src2d, dst2d, w2d, xs)


def _norm_body(dg0_ref, dg1_ref, x_ref, xs_ref, di_ref):
    di = lax.rsqrt(dg0_ref[...] + dg1_ref[...] + (1.0 + 1e-10))
    di_ref[...] = di
    xs_ref[...] = x_ref[...] * di


def _out_body(a0_ref, a1_ref, xs_ref, di_ref, th_ref, o_ref):
    pre = (a0_ref[...] + a1_ref[...] + xs_ref[...]) * di_ref[...]
    o_ref[...] = jnp.dot(pre, th_ref[...], preferred_element_type=jnp.float32)


def kernel(data, edge_list, weight_list, theta):
    n = data.shape[0]
    e0 = edge_list[:, 0].astype(jnp.int32)
    e1 = edge_list[:, 1].astype(jnp.int32)
    e2 = 2 * edge_list.shape[0]
    pad = E2P - e2
    pad_idx = jnp.arange(pad, dtype=jnp.int32) % n
    src = jnp.concatenate([e0, e1, pad_idx])
    dst = jnp.concatenate([e1, e0, pad_idx])
    w2 = jnp.concatenate([weight_list, weight_list,
                          jnp.zeros((pad,), jnp.float32)])
    src2d = src.reshape(IDX_ROWS, 128)
    dst2d = dst.reshape(IDX_ROWS, 128)
    w2d = w2.reshape(IDX_ROWS, 128)

    deg_parts = _degrees(dst2d, w2d)
    d0 = deg_parts[:NACC].reshape(NACC, 1)
    d1 = deg_parts[NP:NP + NACC].reshape(NACC, 1)

    blk = 400
    grid = NACC // blk
    xs, dinv_col = pl.pallas_call(
        _norm_body,
        grid=(grid,),
        in_specs=[
            pl.BlockSpec((blk, 1), lambda i: (i, 0)),
            pl.BlockSpec((blk, 1), lambda i: (i, 0)),
            pl.BlockSpec((blk, D), lambda i: (i, 0)),
        ],
        out_specs=[
            pl.BlockSpec((blk, D), lambda i: (i, 0)),
            pl.BlockSpec((blk, 1), lambda i: (i, 0)),
        ],
        out_shape=[
            jax.ShapeDtypeStruct((NACC, D), jnp.float32),
            jax.ShapeDtypeStruct((NACC, 1), jnp.float32),
        ],
    )(d0, d1, data)

    acc_parts = _message_pass(src2d, dst2d, w2d, xs)

    out = pl.pallas_call(
        _out_body,
        grid=(grid,),
        in_specs=[
            pl.BlockSpec((blk, D), lambda i: (i, 0)),
            pl.BlockSpec((blk, D), lambda i: (i + NACC // 400, 0)),
            pl.BlockSpec((blk, D), lambda i: (i, 0)),
            pl.BlockSpec((blk, 1), lambda i: (i, 0)),
            pl.BlockSpec((D, D), lambda i: (0, 0)),
        ],
        out_specs=pl.BlockSpec((blk, D), lambda i: (i, 0)),
        out_shape=jax.ShapeDtypeStruct((NACC, D), jnp.float32),
    )(acc_parts, acc_parts, xs, dinv_col, theta)
    return out[:n]
